# Initial kernel scaffold; baseline (speedup 1.0000x reference)
import jax
import jax.numpy as jnp
from jax.experimental import pallas as pl


def kernel(x, edge_index, W1, b1, W2, b2, W3, b3, Wf1, bf1, Wf2, bf2):
    raise NotImplementedError("write your pallas kernel here")


# speedup vs baseline: 9.5270x; 9.5270x over previous
import functools

import jax
import jax.numpy as jnp
from jax import lax
from jax.experimental import pallas as pl
from jax.experimental.pallas import tpu as pltpu
from jax.experimental.pallas import tpu_sc as plsc

N = 50000
E = 800000
H = 64
NC, NS = 2, 16
NP = 51200
HALF = NP // 2
ACC_ROWS = HALF + NS
CH = 128
NCHUNKS = E // CH
ROWS_PER_TILE = HALF // NS
DEG_SLICE = NP // NS
ZR = 160

_mesh = plsc.VectorSubcoreMesh(
    core_axis_name="c", subcore_axis_name="s", num_cores=NC, num_subcores=NS)

_f32 = jnp.float32


def _fill1d(ref, n, val):
    v = jnp.full((16,), val, _f32)

    @pl.loop(0, n // 16)
    def _(i):
        ref[pl.ds(i * 16, 16)] = v


@functools.partial(
    pl.kernel,
    out_type=jax.ShapeDtypeStruct((NC, NP), _f32),
    mesh=_mesh,
    compiler_params=pltpu.CompilerParams(use_tc_tiling_on_sc=False),
    scratch_types=[
        pltpu.MemorySpace.VMEM_SHARED((NP,), _f32),
        pltpu.VMEM((CH,), _f32),
        pltpu.VMEM((CH,), jnp.int32),
        pltpu.VMEM((DEG_SLICE,), _f32),
    ],
)
def _deg_kernel(dst_hbm, degp_hbm, acc, ones_v, didx_v, zbuf):
    c = lax.axis_index("c")
    s = lax.axis_index("s")
    wid = c * NS + s
    _fill1d(zbuf, DEG_SLICE, 0.0)
    pltpu.sync_copy(zbuf, acc.at[pl.ds(s * DEG_SLICE, DEG_SLICE)])
    _fill1d(ones_v, CH, 1.0)
    plsc.subcore_barrier()

    nk = (NCHUNKS + NC * NS - 1) // (NC * NS)

    @pl.loop(0, nk)
    def _(k):
        cid = wid + k * (NC * NS)

        @pl.when(cid < NCHUNKS)
        def _():
            pltpu.sync_copy(dst_hbm.at[pl.ds(cid * CH, CH)], didx_v)
            pltpu.sync_copy(ones_v, acc.at[didx_v], add=True)

    plsc.subcore_barrier()
    sl = pl.ds(s * DEG_SLICE, DEG_SLICE)
    pltpu.sync_copy(acc.at[sl], degp_hbm.at[c, sl])


@functools.partial(
    pl.kernel,
    out_type=jax.ShapeDtypeStruct((NP, H), _f32),
    mesh=_mesh,
    compiler_params=pltpu.CompilerParams(use_tc_tiling_on_sc=False),
    scratch_types=[
        pltpu.MemorySpace.VMEM_SHARED((ACC_ROWS, H), _f32),
        pltpu.VMEM((CH,), jnp.int32),
        pltpu.VMEM((CH,), jnp.int32),
        pltpu.VMEM((CH,), jnp.int32),
        pltpu.VMEM((CH, H), _f32),
        pltpu.VMEM((ZR, H), _f32),
    ],
)
def _spmm_kernel(table_hbm, src_hbm, dst_hbm, out_hbm,
                 acc, sidx, didx, didx2, rows, zrows):
    c = lax.axis_index("c")
    s = lax.axis_index("s")
    base = c * HALF
    trash = HALF + s

    z16 = jnp.zeros((16,), _f32)

    @pl.loop(0, ZR)
    def _(i):
        for j in range(H // 16):
            zrows[i, pl.ds(j * 16, 16)] = z16

    for t in range(ROWS_PER_TILE // ZR):
        pltpu.sync_copy(zrows, acc.at[pl.ds(s * ROWS_PER_TILE + t * ZR, ZR)])
    plsc.subcore_barrier()

    nk = (NCHUNKS + NS - 1) // NS

    @pl.loop(0, nk)
    def _(k):
        cid = s + k * NS

        @pl.when(cid < NCHUNKS)
        def _():
            e0 = cid * CH
            pltpu.sync_copy(src_hbm.at[pl.ds(e0, CH)], sidx)
            pltpu.sync_copy(dst_hbm.at[pl.ds(e0, CH)], didx)
            pltpu.sync_copy(table_hbm.at[sidx], rows)
            for i in range(CH // 16):
                d = didx[pl.ds(i * 16, 16)]
                loc = d - base
                ok = (loc >= 0) & (loc < HALF)
                didx2[pl.ds(i * 16, 16)] = jnp.where(ok, loc, trash)
            pltpu.sync_copy(rows, acc.at[didx2], add=True)

    plsc.subcore_barrier()
    r0 = s * ROWS_PER_TILE
    pltpu.sync_copy(acc.at[pl.ds(r0, ROWS_PER_TILE)],
                    out_hbm.at[pl.ds(base + r0, ROWS_PER_TILE)])


_B = 2048
_GRID = (N + _B - 1) // _B


def _tc_prep(degp, x, W1):
    def body(degp_r, x_r, w_r, dinv_o, hws_o):
        d = degp_r[0, :] + degp_r[1, :] + 1.0
        di = lax.rsqrt(d)
        dinv_o[...] = di[:, None]
        hws_o[...] = (
            jnp.dot(x_r[...], w_r[...], preferred_element_type=_f32)
            * di[:, None])

    return pl.pallas_call(
        body,
        grid=(_GRID,),
        in_specs=[
            pl.BlockSpec((2, _B), lambda i: (0, i)),
            pl.BlockSpec((_B, 2), lambda i: (i, 0)),
            pl.BlockSpec((2, H), lambda i: (0, 0)),
        ],
        out_specs=[
            pl.BlockSpec((_B, 1), lambda i: (i, 0)),
            pl.BlockSpec((_B, H), lambda i: (i, 0)),
        ],
        out_shape=[
            jax.ShapeDtypeStruct((N, 1), _f32),
            jax.ShapeDtypeStruct((N, H), _f32),
        ],
    )(degp, x, W1)


def _tc_layer(p, hws, dinv, b, w):
    def body(p_r, hws_r, dinv_r, b_r, w_r, out_o):
        h = jnp.maximum(dinv_r[...] * (p_r[...] + hws_r[...]) + b_r[...], 0.0)
        out_o[...] = (
            jnp.dot(h, w_r[...], preferred_element_type=_f32) * dinv_r[...])

    return pl.pallas_call(
        body,
        grid=(_GRID,),
        in_specs=[
            pl.BlockSpec((_B, H), lambda i: (i, 0)),
            pl.BlockSpec((_B, H), lambda i: (i, 0)),
            pl.BlockSpec((_B, 1), lambda i: (i, 0)),
            pl.BlockSpec((1, H), lambda i: (0, 0)),
            pl.BlockSpec((H, H), lambda i: (0, 0)),
        ],
        out_specs=pl.BlockSpec((_B, H), lambda i: (i, 0)),
        out_shape=jax.ShapeDtypeStruct((N, H), _f32),
    )(p, hws, dinv, b, w)


def _tc_head(p, hws, dinv, b3, Wf1, bf1, Wf2, bf2):
    def body(p_r, hws_r, dinv_r, b3_r, wf1_r, bf1_r, wf2_r, bf2_r, out_o):
        h = jnp.maximum(dinv_r[...] * (p_r[...] + hws_r[...]) + b3_r[...], 0.0)
        f = jnp.maximum(
            jnp.dot(h, wf1_r[...], preferred_element_type=_f32) + bf1_r[...],
            0.0)
        out_o[...] = (
            jnp.dot(f, wf2_r[...], preferred_element_type=_f32) + bf2_r[...])

    return pl.pallas_call(
        body,
        grid=(_GRID,),
        in_specs=[
            pl.BlockSpec((_B, H), lambda i: (i, 0)),
            pl.BlockSpec((_B, H), lambda i: (i, 0)),
            pl.BlockSpec((_B, 1), lambda i: (i, 0)),
            pl.BlockSpec((1, H), lambda i: (0, 0)),
            pl.BlockSpec((H, 32), lambda i: (0, 0)),
            pl.BlockSpec((1, 32), lambda i: (0, 0)),
            pl.BlockSpec((32, 2), lambda i: (0, 0)),
            pl.BlockSpec((1, 2), lambda i: (0, 0)),
        ],
        out_specs=pl.BlockSpec((_B, 2), lambda i: (i, 0)),
        out_shape=jax.ShapeDtypeStruct((N, 2), _f32),
    )(p, hws, dinv, b3, Wf1, bf1, Wf2, bf2)


def kernel(x, edge_index, W1, b1, W2, b2, W3, b3, Wf1, bf1, Wf2, bf2):
    src = edge_index[0]
    dst = edge_index[1]
    degp = _deg_kernel(dst)
    dinv, hw1s = _tc_prep(degp, x, W1)
    p1 = _spmm_kernel(hw1s, src, dst)
    hw2s = _tc_layer(p1[:N], hw1s, dinv, b1.reshape(1, H), W2)
    p2 = _spmm_kernel(hw2s, src, dst)
    hw3s = _tc_layer(p2[:N], hw2s, dinv, b2.reshape(1, H), W3)
    p3 = _spmm_kernel(hw3s, src, dst)
    return _tc_head(p3[:N], hw3s, dinv, b3.reshape(1, H),
                    Wf1, bf1.reshape(1, 32), Wf2, bf2.reshape(1, 2))


# speedup vs baseline: 17.7810x; 1.8664x over previous
import functools

import jax
import jax.numpy as jnp
from jax import lax
from jax.experimental import pallas as pl
from jax.experimental.pallas import tpu as pltpu
from jax.experimental.pallas import tpu_sc as plsc

N = 50000
E = 800000
H = 64
NC, NS = 2, 16
NP = 51200
HALF = NP // 2
ACC_ROWS = HALF + NS
CH = 128
EP = 802816
NCHUNKS = EP // CH
ROWS_PER_TILE = HALF // NS
DEG_SLICE = NP // NS
ZR = 160

_mesh = plsc.VectorSubcoreMesh(
    core_axis_name="c", subcore_axis_name="s", num_cores=NC, num_subcores=NS)

_f32 = jnp.float32


def _fill1d(ref, n, val):
    v = jnp.full((16,), val, _f32)

    @pl.loop(0, n // 16)
    def _(i):
        ref[pl.ds(i * 16, 16)] = v


@functools.partial(
    pl.kernel,
    out_type=jax.ShapeDtypeStruct((NC, NP), _f32),
    mesh=_mesh,
    compiler_params=pltpu.CompilerParams(use_tc_tiling_on_sc=False),
    scratch_types=[
        pltpu.MemorySpace.VMEM_SHARED((NP,), _f32),
        pltpu.VMEM((CH,), _f32),
        pltpu.VMEM((CH,), jnp.int32),
        pltpu.VMEM((CH,), jnp.int32),
        pltpu.VMEM((DEG_SLICE,), _f32),
        pltpu.SemaphoreType.DMA,
        pltpu.SemaphoreType.DMA,
    ],
)
def _deg_kernel(dst_hbm, degp_hbm, acc, ones_v, didx_a, didx_b, zbuf,
                sem_a, sem_b):
    c = lax.axis_index("c")
    s = lax.axis_index("s")
    wid = c * NS + s
    stride = NC * NS
    _fill1d(zbuf, DEG_SLICE, 0.0)
    pltpu.sync_copy(zbuf, acc.at[pl.ds(s * DEG_SLICE, DEG_SLICE)])
    _fill1d(ones_v, CH, 1.0)
    plsc.subcore_barrier()

    nk = NCHUNKS // stride

    def fetch(k, didx, sem):
        pltpu.async_copy(dst_hbm.at[pl.ds((wid + k * stride) * CH, CH)],
                         didx, sem)

    def wait_fetch(didx, sem):
        pltpu.make_async_copy(dst_hbm.at[pl.ds(0, CH)], didx, sem).wait()

    fetch(0, didx_a, sem_a)
    fetch(1, didx_b, sem_b)

    @pl.loop(0, nk // 2)
    def _(j):
        k0 = 2 * j
        wait_fetch(didx_a, sem_a)
        pltpu.sync_copy(ones_v, acc.at[didx_a], add=True)

        @pl.when(k0 + 2 < nk)
        def _():
            fetch(k0 + 2, didx_a, sem_a)

        wait_fetch(didx_b, sem_b)
        pltpu.sync_copy(ones_v, acc.at[didx_b], add=True)

        @pl.when(k0 + 3 < nk)
        def _():
            fetch(k0 + 3, didx_b, sem_b)

    plsc.subcore_barrier()
    sl = pl.ds(s * DEG_SLICE, DEG_SLICE)
    pltpu.sync_copy(acc.at[sl], degp_hbm.at[c, sl])


@functools.partial(
    pl.kernel,
    out_type=jax.ShapeDtypeStruct((NP, H), _f32),
    mesh=_mesh,
    compiler_params=pltpu.CompilerParams(use_tc_tiling_on_sc=False),
    scratch_types=[
        pltpu.MemorySpace.VMEM_SHARED((ACC_ROWS, H), _f32),
        pltpu.VMEM((CH,), jnp.int32),
        pltpu.VMEM((CH,), jnp.int32),
        pltpu.VMEM((CH,), jnp.int32),
        pltpu.VMEM((CH,), jnp.int32),
        pltpu.VMEM((CH,), jnp.int32),
        pltpu.VMEM((CH,), jnp.int32),
        pltpu.VMEM((CH, H), _f32),
        pltpu.VMEM((CH, H), _f32),
        pltpu.VMEM((ZR, H), _f32),
        pltpu.SemaphoreType.DMA,
        pltpu.SemaphoreType.DMA,
        pltpu.SemaphoreType.DMA,
        pltpu.SemaphoreType.DMA,
    ],
)
def _spmm_kernel(table_hbm, src_hbm, dst_hbm, out_hbm,
                 acc, sidx_a, sidx_b, didx_a, didx_b, didx2_a, didx2_b,
                 rows_a, rows_b, zrows, isem_a, isem_b, gsem_a, gsem_b):
    c = lax.axis_index("c")
    s = lax.axis_index("s")
    base = c * HALF
    trash = HALF + s

    z16 = jnp.zeros((16,), _f32)

    @pl.loop(0, ZR)
    def _(i):
        for j in range(H // 16):
            zrows[i, pl.ds(j * 16, 16)] = z16

    for t in range(ROWS_PER_TILE // ZR):
        pltpu.sync_copy(zrows, acc.at[pl.ds(s * ROWS_PER_TILE + t * ZR, ZR)])
    plsc.subcore_barrier()

    nk = NCHUNKS // NS

    def fetch_idx(k, sidx, didx, isem):
        e0 = (s + k * NS) * CH
        pltpu.async_copy(src_hbm.at[pl.ds(e0, CH)], sidx, isem)
        pltpu.async_copy(dst_hbm.at[pl.ds(e0, CH)], didx, isem)

    def wait_idx(sidx, didx, isem):
        dummy = src_hbm.at[pl.ds(0, CH)]
        pltpu.make_async_copy(dummy, sidx, isem).wait()
        pltpu.make_async_copy(dummy, didx, isem).wait()

    def gather(sidx, rows, gsem):
        pltpu.async_copy(table_hbm.at[sidx], rows, gsem)

    def wait_gather(sidx, rows, gsem):
        pltpu.make_async_copy(table_hbm.at[sidx], rows, gsem).wait()

    def localize(didx, didx2):
        for i in range(CH // 16):
            d = didx[pl.ds(i * 16, 16)]
            loc = d - base
            ok = (loc >= 0) & (loc < HALF)
            didx2[pl.ds(i * 16, 16)] = jnp.where(ok, loc, trash)

    fetch_idx(0, sidx_a, didx_a, isem_a)
    fetch_idx(1, sidx_b, didx_b, isem_b)
    wait_idx(sidx_a, didx_a, isem_a)
    gather(sidx_a, rows_a, gsem_a)

    @pl.loop(0, nk // 2)
    def _(j):
        k0 = 2 * j
        localize(didx_a, didx2_a)
        wait_idx(sidx_b, didx_b, isem_b)
        gather(sidx_b, rows_b, gsem_b)
        wait_gather(sidx_a, rows_a, gsem_a)

        @pl.when(k0 + 2 < nk)
        def _():
            fetch_idx(k0 + 2, sidx_a, didx_a, isem_a)

        pltpu.sync_copy(rows_a, acc.at[didx2_a], add=True)
        localize(didx_b, didx2_b)

        @pl.when(k0 + 2 < nk)
        def _():
            wait_idx(sidx_a, didx_a, isem_a)
            gather(sidx_a, rows_a, gsem_a)

        wait_gather(sidx_b, rows_b, gsem_b)

        @pl.when(k0 + 3 < nk)
        def _():
            fetch_idx(k0 + 3, sidx_b, didx_b, isem_b)

        pltpu.sync_copy(rows_b, acc.at[didx2_b], add=True)

    plsc.subcore_barrier()
    r0 = s * ROWS_PER_TILE
    pltpu.sync_copy(acc.at[pl.ds(r0, ROWS_PER_TILE)],
                    out_hbm.at[pl.ds(base + r0, ROWS_PER_TILE)])


_B = 2048
_GRID = (N + _B - 1) // _B


def _tc_prep(degp, x, W1):
    def body(degp_r, x_r, w_r, dinv_o, hws_o):
        d = degp_r[0, :] + degp_r[1, :] + 1.0
        di = lax.rsqrt(d)
        dinv_o[...] = di[:, None]
        hws_o[...] = (
            jnp.dot(x_r[...], w_r[...], preferred_element_type=_f32)
            * di[:, None])

    return pl.pallas_call(
        body,
        grid=(_GRID,),
        in_specs=[
            pl.BlockSpec((2, _B), lambda i: (0, i)),
            pl.BlockSpec((_B, 2), lambda i: (i, 0)),
            pl.BlockSpec((2, H), lambda i: (0, 0)),
        ],
        out_specs=[
            pl.BlockSpec((_B, 1), lambda i: (i, 0)),
            pl.BlockSpec((_B, H), lambda i: (i, 0)),
        ],
        out_shape=[
            jax.ShapeDtypeStruct((N, 1), _f32),
            jax.ShapeDtypeStruct((N, H), _f32),
        ],
    )(degp, x, W1)


def _tc_layer(p, hws, dinv, b, w):
    def body(p_r, hws_r, dinv_r, b_r, w_r, out_o):
        h = jnp.maximum(dinv_r[...] * (p_r[...] + hws_r[...]) + b_r[...], 0.0)
        out_o[...] = (
            jnp.dot(h, w_r[...], preferred_element_type=_f32) * dinv_r[...])

    return pl.pallas_call(
        body,
        grid=(_GRID,),
        in_specs=[
            pl.BlockSpec((_B, H), lambda i: (i, 0)),
            pl.BlockSpec((_B, H), lambda i: (i, 0)),
            pl.BlockSpec((_B, 1), lambda i: (i, 0)),
            pl.BlockSpec((1, H), lambda i: (0, 0)),
            pl.BlockSpec((H, H), lambda i: (0, 0)),
        ],
        out_specs=pl.BlockSpec((_B, H), lambda i: (i, 0)),
        out_shape=jax.ShapeDtypeStruct((N, H), _f32),
    )(p, hws, dinv, b, w)


def _tc_head(p, hws, dinv, b3, Wf1, bf1, Wf2, bf2):
    def body(p_r, hws_r, dinv_r, b3_r, wf1_r, bf1_r, wf2_r, bf2_r, out_o):
        h = jnp.maximum(dinv_r[...] * (p_r[...] + hws_r[...]) + b3_r[...], 0.0)
        f = jnp.maximum(
            jnp.dot(h, wf1_r[...], preferred_element_type=_f32) + bf1_r[...],
            0.0)
        out_o[...] = (
            jnp.dot(f, wf2_r[...], preferred_element_type=_f32) + bf2_r[...])

    return pl.pallas_call(
        body,
        grid=(_GRID,),
        in_specs=[
            pl.BlockSpec((_B, H), lambda i: (i, 0)),
            pl.BlockSpec((_B, H), lambda i: (i, 0)),
            pl.BlockSpec((_B, 1), lambda i: (i, 0)),
            pl.BlockSpec((1, H), lambda i: (0, 0)),
            pl.BlockSpec((H, 32), lambda i: (0, 0)),
            pl.BlockSpec((1, 32), lambda i: (0, 0)),
            pl.BlockSpec((32, 2), lambda i: (0, 0)),
            pl.BlockSpec((1, 2), lambda i: (0, 0)),
        ],
        out_specs=pl.BlockSpec((_B, 2), lambda i: (i, 0)),
        out_shape=jax.ShapeDtypeStruct((N, 2), _f32),
    )(p, hws, dinv, b3, Wf1, bf1, Wf2, bf2)


def kernel(x, edge_index, W1, b1, W2, b2, W3, b3, Wf1, bf1, Wf2, bf2):
    pad = EP - E
    src = jnp.concatenate([edge_index[0], jnp.zeros((pad,), jnp.int32)])
    dst = jnp.concatenate([edge_index[1], jnp.full((pad,), N, jnp.int32)])
    degp = _deg_kernel(dst)
    dinv, hw1s = _tc_prep(degp, x, W1)
    p1 = _spmm_kernel(hw1s, src, dst)
    hw2s = _tc_layer(p1[:N], hw1s, dinv, b1.reshape(1, H), W2)
    p2 = _spmm_kernel(hw2s, src, dst)
    hw3s = _tc_layer(p2[:N], hw2s, dinv, b2.reshape(1, H), W3)
    p3 = _spmm_kernel(hw3s, src, dst)
    return _tc_head(p3[:N], hw3s, dinv, b3.reshape(1, H),
                    Wf1, bf1.reshape(1, 32), Wf2, bf2.reshape(1, 2))
